# baseline (speedup 1.0000x reference)
import functools

import jax
import jax.numpy as jnp
from jax import lax
from jax.experimental import pallas as pl
from jax.experimental.pallas import tpu as pltpu
from jax.experimental.pallas import tpu_sc as plsc


def _proj_body(f_ref, wkT_ref, bk_ref, wqT_ref, bq_ref, ks_ref, qs_ref):
    f = f_ref[...]
    for w_ref, b_ref, o_ref in ((wkT_ref, bk_ref, ks_ref), (wqT_ref, bq_ref, qs_ref)):
        v = jnp.dot(f, w_ref[...], preferred_element_type=jnp.float32) + b_ref[...]
        d = v.shape[1]
        lo = lax.bitcast_convert_type(v[:, : d // 2].astype(jnp.bfloat16), jnp.uint16)
        hi = lax.bitcast_convert_type(v[:, d // 2 :].astype(jnp.bfloat16), jnp.uint16)
        w32 = lo.astype(jnp.uint32) | (hi.astype(jnp.uint32) << 16)
        o_ref[...] = w32.astype(jnp.int32)


def _project_packed(feats, wkT, bk2, wqT, bq2, blk):
    n, latent = feats.shape
    d = wkT.shape[1]
    grid = n // blk
    return pl.pallas_call(
        _proj_body,
        grid=(grid,),
        in_specs=[
            pl.BlockSpec((blk, latent), lambda i: (i, 0)),
            pl.BlockSpec((latent, d), lambda i: (0, 0)),
            pl.BlockSpec((1, d), lambda i: (0, 0)),
            pl.BlockSpec((latent, d), lambda i: (0, 0)),
            pl.BlockSpec((1, d), lambda i: (0, 0)),
        ],
        out_specs=[
            pl.BlockSpec((blk, d // 2), lambda i: (i, 0)),
            pl.BlockSpec((blk, d // 2), lambda i: (i, 0)),
        ],
        out_shape=[
            jax.ShapeDtypeStruct((n, d // 2), jnp.int32),
            jax.ShapeDtypeStruct((n, d // 2), jnp.int32),
        ],
    )(feats, wkT, bk2, wqT, bq2)


def _make_sc_affinity(nk, w, nw, e_blk):
    c_per_w = nk // nw
    n_blocks = c_per_w // e_blk
    n_groups = e_blk // 16

    mesh = plsc.VectorSubcoreMesh(core_axis_name="c", subcore_axis_name="s")
    nc = mesh.num_cores

    @functools.partial(
        pl.kernel,
        mesh=mesh,
        out_type=jax.ShapeDtypeStruct((nk,), jnp.float32),
        scratch_types=[
            [pltpu.VMEM((e_blk,), jnp.int32) for _ in range(2)],
            [pltpu.VMEM((e_blk,), jnp.int32) for _ in range(2)],
            [pltpu.VMEM((e_blk, w), jnp.int32) for _ in range(2)],
            [pltpu.VMEM((e_blk, w), jnp.int32) for _ in range(2)],
            pltpu.VMEM((e_blk,), jnp.float32),
            [pltpu.SemaphoreType.DMA for _ in range(2)],
        ],
        compiler_params=pltpu.CompilerParams(
            needs_layout_passes=False, use_tc_tiling_on_sc=False
        ),
    )
    def sc_kernel(ks_hbm, qs_hbm, idx3_hbm, out_hbm,
                  xidx_v, yidx_v, xrows, yrows, out_v, sems):
        wid = lax.axis_index("s") * nc + lax.axis_index("c")
        base_w = wid * c_per_w

        def issue(bb, i):
            base = base_w + bb * e_blk
            pltpu.sync_copy(idx3_hbm.at[1, pl.ds(base, e_blk)], xidx_v[i])
            pltpu.sync_copy(idx3_hbm.at[2, pl.ds(base, e_blk)], yidx_v[i])
            pltpu.make_async_copy(ks_hbm.at[xidx_v[i]], xrows[i], sems[i]).start()
            pltpu.make_async_copy(qs_hbm.at[yidx_v[i]], yrows[i], sems[i]).start()

        def compute(bb, i):
            base = base_w + bb * e_blk
            pltpu.make_async_copy(ks_hbm.at[xidx_v[i]], xrows[i], sems[i]).wait()
            pltpu.make_async_copy(qs_hbm.at[yidx_v[i]], yrows[i], sems[i]).wait()
            lane = lax.iota(jnp.int32, 16)

            def group_body(g, carry2):
                rowv = g * 16 + lane
                acc = jnp.zeros((16,), jnp.float32)
                for j in range(w):
                    colv = (lane + j) % w
                    xw = plsc.load_gather(xrows[i], [rowv, colv])
                    yw = plsc.load_gather(yrows[i], [rowv, colv])
                    xb = plsc.bitcast(xw, jnp.bfloat16)
                    yb = plsc.bitcast(yw, jnp.bfloat16)
                    pa, pb = plsc.unpack(xb * yb, format=plsc.PackFormat.INTERLEAVED)
                    acc = acc + pa + pb
                out_v[pl.ds(g * 16, 16)] = acc
                return carry2

            lax.fori_loop(0, n_groups, group_body, 0, unroll=False)
            pltpu.sync_copy(out_v, out_hbm.at[pl.ds(base, e_blk)])

        issue(0, 0)

        def pipe_body(t, carry):
            issue(2 * t + 1, 1)
            compute(2 * t, 0)
            issue(2 * t + 2, 0)
            compute(2 * t + 1, 1)
            return carry

        lax.fori_loop(0, (n_blocks - 1) // 2, pipe_body, 0, unroll=False)
        compute(n_blocks - 1, 0)

    return sc_kernel


def kernel(features, Wk, bk, Wq, bq, img, indices):
    del img
    b, n, latent = features.shape
    _, _, _, k = indices.shape
    d = Wk.shape[0]
    feats = features.reshape(b * n, latent)
    scale = jnp.float32(d) ** jnp.float32(-0.5)

    wkT = (Wk.T * scale).astype(jnp.float32)
    wqT = Wq.T.astype(jnp.float32)
    bk2 = (bk * scale).reshape(1, d).astype(jnp.float32)
    bq2 = bq.reshape(1, d).astype(jnp.float32)

    blk = 2000 if (b * n) % 2000 == 0 else 8
    ks, qs = _project_packed(feats, wkT, bk2, wqT, bq2, blk)

    nk = b * n * k
    if b > 1:
        off = (jnp.arange(b, dtype=jnp.int32) * n)[None, :, None]
        idx3 = (indices.reshape(3, b, n * k) + off).reshape(3, nk)
    else:
        idx3 = indices.reshape(3, nk)
    idx3 = idx3.astype(jnp.int32)
    nw = 32
    e_blk = 400
    if (nk % nw) or ((nk // nw) % e_blk) or (e_blk % 16):
        e_blk = 16
    sc_fn = _make_sc_affinity(nk, d // 2, nw, e_blk)
    logits = sc_fn(ks, qs, idx3)
    return logits.reshape(b, n, k)


# speedup vs baseline: 1.0760x; 1.0760x over previous
import functools

import jax
import jax.numpy as jnp
from jax import lax
from jax.experimental import pallas as pl
from jax.experimental.pallas import tpu as pltpu
from jax.experimental.pallas import tpu_sc as plsc


def _proj_body(f_ref, wkT_ref, bk_ref, wqT_ref, bq_ref, ks_ref, qs_ref):
    f = f_ref[...]
    for w_ref, b_ref, o_ref in ((wkT_ref, bk_ref, ks_ref), (wqT_ref, bq_ref, qs_ref)):
        v = jnp.dot(f, w_ref[...], preferred_element_type=jnp.float32) + b_ref[...]
        d = v.shape[1]
        lo = lax.bitcast_convert_type(v[:, : d // 2].astype(jnp.bfloat16), jnp.uint16)
        hi = lax.bitcast_convert_type(v[:, d // 2 :].astype(jnp.bfloat16), jnp.uint16)
        w32 = lo.astype(jnp.uint32) | (hi.astype(jnp.uint32) << 16)
        o_ref[...] = w32.astype(jnp.int32)


def _project_packed(feats, wkT, bk2, wqT, bq2, blk):
    n, latent = feats.shape
    d = wkT.shape[1]
    grid = n // blk
    return pl.pallas_call(
        _proj_body,
        grid=(grid,),
        in_specs=[
            pl.BlockSpec((blk, latent), lambda i: (i, 0)),
            pl.BlockSpec((latent, d), lambda i: (0, 0)),
            pl.BlockSpec((1, d), lambda i: (0, 0)),
            pl.BlockSpec((latent, d), lambda i: (0, 0)),
            pl.BlockSpec((1, d), lambda i: (0, 0)),
        ],
        out_specs=[
            pl.BlockSpec((blk, d // 2), lambda i: (i, 0)),
            pl.BlockSpec((blk, d // 2), lambda i: (i, 0)),
        ],
        out_shape=[
            jax.ShapeDtypeStruct((n, d // 2), jnp.int32),
            jax.ShapeDtypeStruct((n, d // 2), jnp.int32),
        ],
    )(feats, wkT, bk2, wqT, bq2)


def _make_sc_affinity(nk, n_rows, w, nw, e_blk):
    c_per_w = nk // nw
    n_blocks = c_per_w // e_blk
    n_groups = e_blk // 16

    mesh = plsc.VectorSubcoreMesh(core_axis_name="c", subcore_axis_name="s")
    nc = mesh.num_cores

    @functools.partial(
        pl.kernel,
        mesh=mesh,
        out_type=jax.ShapeDtypeStruct((nk,), jnp.float32),
        scratch_types=[
            [pltpu.VMEM((e_blk,), jnp.int32) for _ in range(2)],
            [pltpu.VMEM((e_blk,), jnp.int32) for _ in range(2)],
            [pltpu.VMEM((e_blk, w), jnp.int32) for _ in range(2)],
            [pltpu.VMEM((e_blk, w), jnp.int32) for _ in range(2)],
            pltpu.VMEM((e_blk,), jnp.float32),
            [pltpu.SemaphoreType.DMA for _ in range(2)],
            pltpu.VMEM_SHARED((n_rows, w), jnp.int32),
            pltpu.VMEM_SHARED((n_rows, w), jnp.int32),
        ],
        compiler_params=pltpu.CompilerParams(
            needs_layout_passes=False, use_tc_tiling_on_sc=False
        ),
    )
    def sc_kernel(ks_hbm, qs_hbm, idx3_hbm, out_hbm,
                  xidx_v, yidx_v, xrows, yrows, out_v, sems, ksh, qsh):
        sid = lax.axis_index("s")
        wid = sid * nc + lax.axis_index("c")
        base_w = wid * c_per_w

        @pl.when(sid == 0)
        def _stage_k():
            pltpu.sync_copy(ks_hbm, ksh)

        @pl.when(sid == 1)
        def _stage_q():
            pltpu.sync_copy(qs_hbm, qsh)

        plsc.subcore_barrier()

        def issue(bb, i):
            base = base_w + bb * e_blk
            pltpu.sync_copy(idx3_hbm.at[1, pl.ds(base, e_blk)], xidx_v[i])
            pltpu.sync_copy(idx3_hbm.at[2, pl.ds(base, e_blk)], yidx_v[i])
            pltpu.make_async_copy(ksh.at[xidx_v[i]], xrows[i], sems[i]).start()
            pltpu.make_async_copy(qsh.at[yidx_v[i]], yrows[i], sems[i]).start()

        def compute(bb, i):
            base = base_w + bb * e_blk
            pltpu.make_async_copy(ksh.at[xidx_v[i]], xrows[i], sems[i]).wait()
            pltpu.make_async_copy(qsh.at[yidx_v[i]], yrows[i], sems[i]).wait()
            lane = lax.iota(jnp.int32, 16)

            def group_body(g, carry2):
                rowv = g * 16 + lane
                accs = [jnp.zeros((16,), jnp.float32) for _ in range(4)]
                for j in range(w):
                    colv = (lane + j) % w
                    xw = plsc.load_gather(xrows[i], [rowv, colv])
                    yw = plsc.load_gather(yrows[i], [rowv, colv])
                    xb = plsc.bitcast(xw, jnp.bfloat16)
                    yb = plsc.bitcast(yw, jnp.bfloat16)
                    pa, pb = plsc.unpack(xb * yb, format=plsc.PackFormat.INTERLEAVED)
                    accs[j % 4] = accs[j % 4] + pa + pb
                acc = (accs[0] + accs[1]) + (accs[2] + accs[3])
                out_v[pl.ds(g * 16, 16)] = acc
                return carry2

            lax.fori_loop(0, n_groups, group_body, 0, unroll=False)
            pltpu.sync_copy(out_v, out_hbm.at[pl.ds(base, e_blk)])

        issue(0, 0)

        def pipe_body(t, carry):
            issue(2 * t + 1, 1)
            compute(2 * t, 0)
            issue(2 * t + 2, 0)
            compute(2 * t + 1, 1)
            return carry

        lax.fori_loop(0, (n_blocks - 1) // 2, pipe_body, 0, unroll=False)
        compute(n_blocks - 1, 0)

    return sc_kernel


def kernel(features, Wk, bk, Wq, bq, img, indices):
    del img
    b, n, latent = features.shape
    _, _, _, k = indices.shape
    d = Wk.shape[0]
    feats = features.reshape(b * n, latent)
    scale = jnp.float32(d) ** jnp.float32(-0.5)

    wkT = (Wk.T * scale).astype(jnp.float32)
    wqT = Wq.T.astype(jnp.float32)
    bk2 = (bk * scale).reshape(1, d).astype(jnp.float32)
    bq2 = bq.reshape(1, d).astype(jnp.float32)

    blk = 2000 if (b * n) % 2000 == 0 else 8
    ks, qs = _project_packed(feats, wkT, bk2, wqT, bq2, blk)

    nk = b * n * k
    if b > 1:
        off = (jnp.arange(b, dtype=jnp.int32) * n)[None, :, None]
        idx3 = (indices.reshape(3, b, n * k) + off).reshape(3, nk)
    else:
        idx3 = indices.reshape(3, nk)
    idx3 = idx3.astype(jnp.int32)
    nw = 32
    e_blk = 400
    if (nk % nw) or ((nk // nw) % e_blk) or (e_blk % 16):
        e_blk = 16
    sc_fn = _make_sc_affinity(nk, b * n, d // 2, nw, e_blk)
    logits = sc_fn(ks, qs, idx3)
    return logits.reshape(b, n, k)


# speedup vs baseline: 1.0864x; 1.0097x over previous
import functools

import jax
import jax.numpy as jnp
from jax import lax
from jax.experimental import pallas as pl
from jax.experimental.pallas import tpu as pltpu
from jax.experimental.pallas import tpu_sc as plsc


def _proj_body(f_ref, wkT_ref, bk_ref, wqT_ref, bq_ref, tab_ref):
    f = f_ref[...]
    for t, (w_ref, b_ref) in enumerate(((wkT_ref, bk_ref), (wqT_ref, bq_ref))):
        v = jnp.dot(f, w_ref[...], preferred_element_type=jnp.float32) + b_ref[...]
        d = v.shape[1]
        lo = lax.bitcast_convert_type(v[:, : d // 2].astype(jnp.bfloat16), jnp.uint16)
        hi = lax.bitcast_convert_type(v[:, d // 2 :].astype(jnp.bfloat16), jnp.uint16)
        w32 = lo.astype(jnp.uint32) | (hi.astype(jnp.uint32) << 16)
        tab_ref[t, :, :] = w32.astype(jnp.int32)


def _project_packed(feats, wkT, bk2, wqT, bq2, blk):
    n, latent = feats.shape
    d = wkT.shape[1]
    grid = n // blk
    return pl.pallas_call(
        _proj_body,
        grid=(grid,),
        in_specs=[
            pl.BlockSpec((blk, latent), lambda i: (i, 0)),
            pl.BlockSpec((latent, d), lambda i: (0, 0)),
            pl.BlockSpec((1, d), lambda i: (0, 0)),
            pl.BlockSpec((latent, d), lambda i: (0, 0)),
            pl.BlockSpec((1, d), lambda i: (0, 0)),
        ],
        out_specs=pl.BlockSpec((2, blk, d // 2), lambda i: (0, i, 0)),
        out_shape=jax.ShapeDtypeStruct((2, n, d // 2), jnp.int32),
    )(feats, wkT, bk2, wqT, bq2)


def _make_sc_affinity(nk, n_rows, w, nw, e_blk):
    c_per_w = nk // nw
    n_blocks = c_per_w // e_blk
    n_groups = e_blk // 16

    mesh = plsc.VectorSubcoreMesh(core_axis_name="c", subcore_axis_name="s")
    nc = mesh.num_cores

    @functools.partial(
        pl.kernel,
        mesh=mesh,
        out_type=jax.ShapeDtypeStruct((nk,), jnp.float32),
        scratch_types=[
            [pltpu.VMEM((e_blk,), jnp.int32) for _ in range(2)],
            [pltpu.VMEM((e_blk,), jnp.int32) for _ in range(2)],
            [pltpu.VMEM((e_blk, w), jnp.int32) for _ in range(2)],
            [pltpu.VMEM((e_blk, w), jnp.int32) for _ in range(2)],
            pltpu.VMEM((e_blk,), jnp.float32),
            [pltpu.SemaphoreType.DMA for _ in range(2)],
            pltpu.VMEM_SHARED((n_rows, w), jnp.int32),
            pltpu.VMEM_SHARED((n_rows, w), jnp.int32),
        ],
        compiler_params=pltpu.CompilerParams(
            needs_layout_passes=False, use_tc_tiling_on_sc=False
        ),
    )
    def sc_kernel(tab_hbm, idx3_hbm, out_hbm,
                  xidx_v, yidx_v, xrows, yrows, out_v, sems, ksh, qsh):
        sid = lax.axis_index("s")
        wid = sid * nc + lax.axis_index("c")
        base_w = wid * c_per_w

        @pl.when(sid == 0)
        def _stage_k():
            pltpu.sync_copy(tab_hbm.at[0], ksh)

        @pl.when(sid == 1)
        def _stage_q():
            pltpu.sync_copy(tab_hbm.at[1], qsh)

        plsc.subcore_barrier()

        def issue(bb, i):
            base = base_w + bb * e_blk
            pltpu.sync_copy(idx3_hbm.at[1, pl.ds(base, e_blk)], xidx_v[i])
            pltpu.sync_copy(idx3_hbm.at[2, pl.ds(base, e_blk)], yidx_v[i])
            pltpu.make_async_copy(ksh.at[xidx_v[i]], xrows[i], sems[i]).start()
            pltpu.make_async_copy(qsh.at[yidx_v[i]], yrows[i], sems[i]).start()

        def compute(bb, i):
            base = base_w + bb * e_blk
            pltpu.make_async_copy(ksh.at[xidx_v[i]], xrows[i], sems[i]).wait()
            pltpu.make_async_copy(qsh.at[yidx_v[i]], yrows[i], sems[i]).wait()
            lane = lax.iota(jnp.int32, 16)

            def group_body(g, carry2):
                rowv = g * 16 + lane
                accs = [jnp.zeros((16,), jnp.float32) for _ in range(4)]
                for j in range(w):
                    colv = (lane + j) % w
                    xw = plsc.load_gather(xrows[i], [rowv, colv])
                    yw = plsc.load_gather(yrows[i], [rowv, colv])
                    xb = plsc.bitcast(xw, jnp.bfloat16)
                    yb = plsc.bitcast(yw, jnp.bfloat16)
                    pa, pb = plsc.unpack(xb * yb, format=plsc.PackFormat.INTERLEAVED)
                    accs[j % 4] = accs[j % 4] + pa + pb
                acc = (accs[0] + accs[1]) + (accs[2] + accs[3])
                out_v[pl.ds(g * 16, 16)] = acc
                return carry2

            lax.fori_loop(0, n_groups, group_body, 0, unroll=False)
            pltpu.sync_copy(out_v, out_hbm.at[pl.ds(base, e_blk)])

        issue(0, 0)

        def pipe_body(t, carry):
            issue(2 * t + 1, 1)
            compute(2 * t, 0)
            issue(2 * t + 2, 0)
            compute(2 * t + 1, 1)
            return carry

        lax.fori_loop(0, (n_blocks - 1) // 2, pipe_body, 0, unroll=False)
        compute(n_blocks - 1, 0)

    return sc_kernel


def kernel(features, Wk, bk, Wq, bq, img, indices):
    del img
    b, n, latent = features.shape
    _, _, _, k = indices.shape
    d = Wk.shape[0]
    feats = features.reshape(b * n, latent)
    scale = jnp.float32(d) ** jnp.float32(-0.5)

    wkT = (Wk.T * scale).astype(jnp.float32)
    wqT = Wq.T.astype(jnp.float32)
    bk2 = (bk * scale).reshape(1, d).astype(jnp.float32)
    bq2 = bq.reshape(1, d).astype(jnp.float32)

    blk = 2000 if (b * n) % 2000 == 0 else 8
    tab = _project_packed(feats, wkT, bk2, wqT, bq2, blk)

    nk = b * n * k
    if b > 1:
        off = (jnp.arange(b, dtype=jnp.int32) * n)[None, :, None]
        idx3 = (indices.reshape(3, b, n * k) + off).reshape(3, nk)
    else:
        idx3 = indices.reshape(3, nk)
    idx3 = idx3.astype(jnp.int32)
    nw = 32
    e_blk = 400
    if (nk % nw) or ((nk // nw) % e_blk) or (e_blk % 16):
        e_blk = 16
    sc_fn = _make_sc_affinity(nk, b * n, d // 2, nw, e_blk)
    logits = sc_fn(tab, idx3)
    return logits.reshape(b, n, k)


# speedup vs baseline: 1.3047x; 1.2009x over previous
import functools

import jax
import jax.numpy as jnp
from jax import lax
from jax.experimental import pallas as pl
from jax.experimental.pallas import tpu as pltpu
from jax.experimental.pallas import tpu_sc as plsc


def _proj_body(f_ref, wkT_ref, bk_ref, wqT_ref, bq_ref, tab_ref):
    f = f_ref[...]
    for t, (w_ref, b_ref) in enumerate(((wkT_ref, bk_ref), (wqT_ref, bq_ref))):
        v = jnp.dot(f, w_ref[...], preferred_element_type=jnp.float32) + b_ref[...]
        d = v.shape[1]
        lo = lax.bitcast_convert_type(v[:, : d // 2].astype(jnp.bfloat16), jnp.uint16)
        hi = lax.bitcast_convert_type(v[:, d // 2 :].astype(jnp.bfloat16), jnp.uint16)
        w32 = lo.astype(jnp.uint32) | (hi.astype(jnp.uint32) << 16)
        tab_ref[t, :, :] = w32.astype(jnp.int32)


def _project_packed(feats, wkT, bk2, wqT, bq2, blk):
    n, latent = feats.shape
    d = wkT.shape[1]
    grid = n // blk
    return pl.pallas_call(
        _proj_body,
        grid=(grid,),
        in_specs=[
            pl.BlockSpec((blk, latent), lambda i: (i, 0)),
            pl.BlockSpec((latent, d), lambda i: (0, 0)),
            pl.BlockSpec((1, d), lambda i: (0, 0)),
            pl.BlockSpec((latent, d), lambda i: (0, 0)),
            pl.BlockSpec((1, d), lambda i: (0, 0)),
        ],
        out_specs=pl.BlockSpec((2, blk, d // 2), lambda i: (0, i, 0)),
        out_shape=jax.ShapeDtypeStruct((2, n, d // 2), jnp.int32),
    )(feats, wkT, bk2, wqT, bq2)


def _make_sc_affinity(nk, n_rows, w, nw, e_blk):
    c_per_w = nk // nw
    n_blocks = c_per_w // e_blk
    n_groups = e_blk // 16
    ch_b = 5 if n_blocks % 5 == 0 else 1
    n_chunks = n_blocks // ch_b

    mesh = plsc.VectorSubcoreMesh(core_axis_name="c", subcore_axis_name="s")
    nc = mesh.num_cores

    @functools.partial(
        pl.kernel,
        mesh=mesh,
        out_type=jax.ShapeDtypeStruct((nk,), jnp.float32),
        scratch_types=[
            pltpu.VMEM((ch_b * e_blk,), jnp.int32),
            pltpu.VMEM((ch_b * e_blk,), jnp.int32),
            [pltpu.VMEM((e_blk, w), jnp.int32) for _ in range(2)],
            [pltpu.VMEM((e_blk, w), jnp.int32) for _ in range(2)],
            pltpu.VMEM((e_blk,), jnp.float32),
            [pltpu.SemaphoreType.DMA for _ in range(2)],
            pltpu.VMEM_SHARED((n_rows, w), jnp.int32),
            pltpu.VMEM_SHARED((n_rows, w), jnp.int32),
        ],
        compiler_params=pltpu.CompilerParams(
            needs_layout_passes=False, use_tc_tiling_on_sc=False
        ),
    )
    def sc_kernel(tab_hbm, idx3_hbm, out_hbm,
                  xidx_v, yidx_v, xrows, yrows, out_v, sems, ksh, qsh):
        sid = lax.axis_index("s")
        wid = sid * nc + lax.axis_index("c")
        base_w = wid * c_per_w

        @pl.when(sid == 0)
        def _stage_k():
            pltpu.sync_copy(tab_hbm.at[0], ksh)

        @pl.when(sid == 1)
        def _stage_q():
            pltpu.sync_copy(tab_hbm.at[1], qsh)

        plsc.subcore_barrier()

        def issue(bb, i):
            xslice = xidx_v.at[pl.ds(bb * e_blk, e_blk)]
            yslice = yidx_v.at[pl.ds(bb * e_blk, e_blk)]
            pltpu.make_async_copy(ksh.at[xslice], xrows[i], sems[i]).start()
            pltpu.make_async_copy(qsh.at[yslice], yrows[i], sems[i]).start()

        def compute(bb, i):
            xslice = xidx_v.at[pl.ds(bb * e_blk, e_blk)]
            yslice = yidx_v.at[pl.ds(bb * e_blk, e_blk)]
            pltpu.make_async_copy(ksh.at[xslice], xrows[i], sems[i]).wait()
            pltpu.make_async_copy(qsh.at[yslice], yrows[i], sems[i]).wait()
            lane = lax.iota(jnp.int32, 16)

            def group_body(g, carry2):
                rowv = g * 16 + lane
                accs = [jnp.zeros((16,), jnp.float32) for _ in range(4)]
                for j in range(w):
                    colv = (lane + j) % w
                    xw = plsc.load_gather(xrows[i], [rowv, colv])
                    yw = plsc.load_gather(yrows[i], [rowv, colv])
                    xb = plsc.bitcast(xw, jnp.bfloat16)
                    yb = plsc.bitcast(yw, jnp.bfloat16)
                    pa, pb = plsc.unpack(xb * yb, format=plsc.PackFormat.INTERLEAVED)
                    accs[j % 4] = accs[j % 4] + pa + pb
                acc = (accs[0] + accs[1]) + (accs[2] + accs[3])
                out_v[pl.ds(g * 16, 16)] = acc
                return carry2

            lax.fori_loop(0, n_groups, group_body, 0, unroll=False)

        def chunk_body(ch, carry):
            chbase = base_w + ch * (ch_b * e_blk)

            def compute_store(bb, i):
                compute(bb, i)
                pltpu.sync_copy(out_v, out_hbm.at[pl.ds(chbase + bb * e_blk, e_blk)])
            pltpu.sync_copy(idx3_hbm.at[1, pl.ds(chbase, ch_b * e_blk)], xidx_v)
            pltpu.sync_copy(idx3_hbm.at[2, pl.ds(chbase, ch_b * e_blk)], yidx_v)

            issue(0, 0)

            def pipe_body(t, carry2):
                issue(2 * t + 1, 1)
                compute_store(2 * t, 0)
                issue(2 * t + 2, 0)
                compute_store(2 * t + 1, 1)
                return carry2

            lax.fori_loop(0, (ch_b - 1) // 2, pipe_body, 0, unroll=False)
            compute_store(ch_b - 1, 0)
            return carry

        lax.fori_loop(0, n_chunks, chunk_body, 0, unroll=False)

    return sc_kernel


def kernel(features, Wk, bk, Wq, bq, img, indices):
    del img
    b, n, latent = features.shape
    _, _, _, k = indices.shape
    d = Wk.shape[0]
    feats = features.reshape(b * n, latent)
    scale = jnp.float32(d) ** jnp.float32(-0.5)

    wkT = (Wk.T * scale).astype(jnp.float32)
    wqT = Wq.T.astype(jnp.float32)
    bk2 = (bk * scale).reshape(1, d).astype(jnp.float32)
    bq2 = bq.reshape(1, d).astype(jnp.float32)

    blk = 2000 if (b * n) % 2000 == 0 else 8
    tab = _project_packed(feats, wkT, bk2, wqT, bq2, blk)

    nk = b * n * k
    if b > 1:
        off = (jnp.arange(b, dtype=jnp.int32) * n)[None, :, None]
        idx3 = (indices.reshape(3, b, n * k) + off).reshape(3, nk)
    else:
        idx3 = indices.reshape(3, nk)
    idx3 = idx3.astype(jnp.int32)
    nw = 32
    e_blk = 400
    if (nk % nw) or ((nk // nw) % e_blk) or (e_blk % 16):
        e_blk = 16
    sc_fn = _make_sc_affinity(nk, b * n, d // 2, nw, e_blk)
    logits = sc_fn(tab, idx3)
    return logits.reshape(b, n, k)
